# baseline (speedup 1.0000x reference)
import functools

import jax
import jax.numpy as jnp
from jax import lax
from jax.experimental import pallas as pl
from jax.experimental.pallas import tpu as pltpu
from jax.experimental.pallas import tpu_sc as plsc

N = 50000
E = 1600000
EA = 16
H = 64
HH = 32
L = 32
C = 6
G = 128

BE = 2560
NBLKA = E // BE


QQ = 16


def _eproj_body(eat_ref, w1t_ref, b1_ref, w2_ref, b2_ref, e1_ref, *e2q_refs):
    eat = eat_ref[...]
    e2 = lax.dot_general(eat, w2_ref[...], (((0,), (0,)), ((), ())),
                         preferred_element_type=jnp.float32) + b2_ref[...]
    for q in range(4):
        e2q_refs[q][...] = e2[:, q * QQ:(q + 1) * QQ]
    e1 = lax.dot_general(w1t_ref[...], eat, (((1,), (0,)), ((), ())),
                         preferred_element_type=jnp.float32) + b1_ref[...]
    e1_ref[...] = e1.reshape(1, 1, BE)


def _edge_proj(eat, We1, be1, We2, be2):
    return pl.pallas_call(
        _eproj_body,
        grid=(NBLKA,),
        in_specs=[
            pl.BlockSpec((EA, BE), lambda i: (0, i)),
            pl.BlockSpec((1, EA), lambda i: (0, 0)),
            pl.BlockSpec((1, 1), lambda i: (0, 0)),
            pl.BlockSpec((EA, H), lambda i: (0, 0)),
            pl.BlockSpec((1, H), lambda i: (0, 0)),
        ],
        out_specs=[pl.BlockSpec((1, 1, BE), lambda i: (i, 0, 0))]
        + [pl.BlockSpec((BE, QQ), lambda i: (i, 0)) for _ in range(4)],
        out_shape=[jax.ShapeDtypeStruct((NBLKA, 1, BE), jnp.float32)]
        + [jax.ShapeDtypeStruct((E, QQ), jnp.float32) for _ in range(4)],
    )(eat, We1.reshape(1, EA), be1.reshape(1, 1), We2, be2.reshape(1, H))


CH1 = 2560
NCH1 = E // CH1
ZPT1 = 3200
NPAD1 = 16 * ZPT1


def _sc_layer1(x, src, dst, e1flat):
    mesh = plsc.VectorSubcoreMesh(core_axis_name="c", subcore_axis_name="s")

    @functools.partial(
        pl.kernel,
        out_type=jax.ShapeDtypeStruct((2, NPAD1), jnp.float32),
        mesh=mesh,
        scratch_types=[
            pltpu.VMEM((N,), jnp.float32),
            pltpu.VMEM((CH1,), jnp.int32),
            pltpu.VMEM((CH1,), jnp.int32),
            pltpu.VMEM((CH1,), jnp.float32),
            pltpu.VMEM((CH1,), jnp.float32),
            pltpu.VMEM((ZPT1,), jnp.float32),
            pltpu.VMEM_SHARED((NPAD1,), jnp.float32),
            pltpu.SemaphoreType.DMA,
        ],
        compiler_params=pltpu.CompilerParams(needs_layout_passes=False, use_tc_tiling_on_sc=False),
    )
    def bkern(x_hbm, src_hbm, dst_hbm, e1_hbm, out_hbm,
              x_v, src_v, dst_v, e1_v, msg_v, zb_v, acc_s, sem):
        c = lax.axis_index("c")
        s = lax.axis_index("s")
        w = s * 2 + c
        pltpu.sync_copy(x_hbm, x_v)

        def zrow(r, _):
            zb_v[pl.ds(r * 16, 16)] = jnp.zeros((16,), jnp.float32)
            return 0

        lax.fori_loop(0, ZPT1 // 16, zrow, 0, unroll=8)
        pltpu.sync_copy(zb_v, acc_s.at[pl.ds(s * ZPT1, ZPT1)])
        plsc.subcore_barrier()

        def chunk(j, _):
            t = w + j * 32

            @pl.when(t < NCH1)
            def _():
                base = t * CH1
                pltpu.sync_copy(src_hbm.at[pl.ds(base, CH1)], src_v)
                pltpu.sync_copy(dst_hbm.at[pl.ds(base, CH1)], dst_v)
                pltpu.sync_copy(e1_hbm.at[pl.ds(base, CH1)], e1_v)

                def row(r, _):
                    for k in range(5):
                        sl = pl.ds((r * 5 + k) * 16, 16)
                        g = plsc.load_gather(x_v, [src_v[sl]])
                        msg_v[sl] = jnp.maximum(g + e1_v[sl], 0.0)
                    return 0

                lax.fori_loop(0, CH1 // 80, row, 0, unroll=4)
                pltpu.sync_copy(msg_v, acc_s.at[dst_v], add=True)

            return 0

        lax.fori_loop(0, (NCH1 + 31) // 32, chunk, 0, unroll=1)
        plsc.subcore_barrier()

        lo = s * ZPT1
        pltpu.sync_copy(acc_s.at[pl.ds(lo, ZPT1)], x_v.at[pl.ds(0, ZPT1)])

        @pl.when(c == 0)
        def _():
            pltpu.sync_copy(x_v.at[pl.ds(0, ZPT1)], out_hbm.at[0].at[pl.ds(lo, ZPT1)])

        @pl.when(c == 1)
        def _():
            pltpu.sync_copy(x_v.at[pl.ds(0, ZPT1)], out_hbm.at[1].at[pl.ds(lo, ZPT1)])

    return bkern(x, src, dst, e1flat)


BN = 2048
NBLK = (N + BN - 1) // BN


def _mlp1_body(x_ref, a_ref, w1a_ref, b1a_ref, w1b_ref, b1b_ref, *h1q_refs):
    h = x_ref[...] + a_ref[0] + a_ref[1]
    t = jnp.maximum(h * w1a_ref[...] + b1a_ref[...], 0.0)
    h1 = jnp.dot(t, w1b_ref[...], preferred_element_type=jnp.float32) + b1b_ref[...]
    h1 = jnp.maximum(h1, 0.0)
    for q in range(4):
        h1q_refs[q][...] = h1[:, q * QQ:(q + 1) * QQ]


def _mlp1(x2, agg1p3, W1a, b1a, W1b, b1b):
    return pl.pallas_call(
        _mlp1_body,
        grid=(NBLK,),
        in_specs=[
            pl.BlockSpec((BN, 1), lambda i: (i, 0)),
            pl.BlockSpec((2, BN, 1), lambda i: (0, i, 0)),
            pl.BlockSpec((1, H), lambda i: (0, 0)),
            pl.BlockSpec((1, H), lambda i: (0, 0)),
            pl.BlockSpec((H, H), lambda i: (0, 0)),
            pl.BlockSpec((1, H), lambda i: (0, 0)),
        ],
        out_specs=[pl.BlockSpec((BN, QQ), lambda i: (i, 0)) for _ in range(4)],
        out_shape=[jax.ShapeDtypeStruct((N, QQ), jnp.float32) for _ in range(4)],
    )(x2, agg1p3, W1a, b1a.reshape(1, H), W1b, b1b.reshape(1, H))


CH2 = 1280
NCH2 = E // CH2
RPT = 3200
NPAD2 = 16 * RPT


def _sc_layer2(h1q, src, dst, e2q):
    mesh = plsc.VectorSubcoreMesh(core_axis_name="c", subcore_axis_name="s")

    @functools.partial(
        pl.kernel,
        out_type=[jax.ShapeDtypeStruct((NPAD2, QQ), jnp.float32)
                  for _ in range(4)],
        mesh=mesh,
        scratch_types=[
            pltpu.VMEM((CH2,), jnp.int32),
            pltpu.VMEM((CH2,), jnp.int32),
            pltpu.VMEM((CH2, QQ), jnp.float32),
            pltpu.VMEM((CH2, QQ), jnp.float32),
            pltpu.VMEM_SHARED((NPAD2, QQ), jnp.float32),
            pltpu.SemaphoreType.DMA,
        ],
        compiler_params=pltpu.CompilerParams(needs_layout_passes=False,
                                             use_tc_tiling_on_sc=False),
    )
    def dkern(h0_hbm, h1_hbm, h2_hbm, h3_hbm, src_hbm, dst_hbm,
              e0_hbm, e1_hbm, e2_hbm, e3_hbm, o0_hbm, o1_hbm, o2_hbm, o3_hbm,
              src_v, dst_v, e2_v, gath_v, acc_s, sem):
        c = lax.axis_index("c")
        s = lax.axis_index("s")
        hq = (h0_hbm, h1_hbm, h2_hbm, h3_hbm)
        eq = (e0_hbm, e1_hbm, e2_hbm, e3_hbm)
        oq = (o0_hbm, o1_hbm, o2_hbm, o3_hbm)

        def zero_acc():
            def zrow(r, _):
                e2_v[r, pl.ds(0, 16)] = jnp.zeros((16,), jnp.float32)
                return 0

            lax.fori_loop(0, CH2, zrow, 0, unroll=8)
            for k in range(2):
                pltpu.sync_copy(e2_v, acc_s.at[pl.ds(s * RPT + k * CH2, CH2)])
            pltpu.sync_copy(e2_v.at[pl.ds(0, RPT - 2 * CH2)],
                            acc_s.at[pl.ds(s * RPT + 2 * CH2, RPT - 2 * CH2)])

        def qpass(h1_hbm, e2_hbm):
            def chunk(j, _):
                t = s + j * 16

                @pl.when(t < NCH2)
                def _():
                    base = t * CH2
                    pltpu.sync_copy(src_hbm.at[pl.ds(base, CH2)], src_v)
                    pltpu.sync_copy(dst_hbm.at[pl.ds(base, CH2)], dst_v)
                    pltpu.sync_copy(e2_hbm.at[pl.ds(base, CH2)], e2_v)
                    pltpu.async_copy(h1_hbm.at[src_v], gath_v, sem).wait()

                    def row(r, _):
                        for k in range(4):
                            rr = r * 4 + k
                            a0 = gath_v[rr, pl.ds(0, 16)] + e2_v[rr, pl.ds(0, 16)]
                            gath_v[rr, pl.ds(0, 16)] = jnp.maximum(a0, 0.0)
                        return 0

                    lax.fori_loop(0, CH2 // 4, row, 0, unroll=8)
                    pltpu.sync_copy(gath_v, acc_s.at[dst_v], add=True)

                return 0

            lax.fori_loop(0, (NCH2 + 15) // 16, chunk, 0, unroll=1)

        def copyout(out_hbm):
            for k in range(3):
                ln = CH2 if k < 2 else RPT - 2 * CH2
                off = s * RPT + k * CH2
                pltpu.sync_copy(acc_s.at[pl.ds(off, ln)], gath_v.at[pl.ds(0, ln)])
                pltpu.sync_copy(gath_v.at[pl.ds(0, ln)], out_hbm.at[pl.ds(off, ln)])

        for p in range(2):
            zero_acc()
            plsc.subcore_barrier()

            @pl.when(c == 0)
            def _():
                qpass(hq[p], eq[p])

            @pl.when(c == 1)
            def _():
                qpass(hq[2 + p], eq[2 + p])

            plsc.subcore_barrier()

            @pl.when(c == 0)
            def _():
                copyout(oq[p])

            @pl.when(c == 1)
            def _():
                copyout(oq[2 + p])

            if p == 0:
                plsc.subcore_barrier()

    return dkern(*h1q, src, dst, *e2q)


def _mlp2_body(h0_ref, h1_ref, h2q_ref, h3_ref, a0_ref, a1_ref, a2_ref, a3_ref,
               batch_ref, eps_ref,
               w2a_ref, b2a_ref, w2b_ref, b2b_ref, wmu_ref, bmu_ref,
               wlv_ref, blv_ref, wc_ref, bc_ref,
               z_ref, mu_ref, lv_ref, logits_ref, psum, pcnt):
    pid = pl.program_id(0)

    @pl.when(pid == 0)
    def _():
        psum[...] = jnp.zeros_like(psum)
        pcnt[...] = jnp.zeros_like(pcnt)

    h1 = jnp.concatenate([h0_ref[...], h1_ref[...], h2q_ref[...], h3_ref[...]],
                         axis=1)
    agg = jnp.concatenate([a0_ref[...], a1_ref[...], a2_ref[...], a3_ref[...]],
                          axis=1)
    h = h1 + agg
    t = jnp.maximum(jnp.dot(h, w2a_ref[...], preferred_element_type=jnp.float32)
                    + b2a_ref[...], 0.0)
    h2 = jnp.dot(t, w2b_ref[...], preferred_element_type=jnp.float32) + b2b_ref[...]
    h2 = jnp.maximum(h2, 0.0)
    mu = jnp.dot(h2, wmu_ref[...], preferred_element_type=jnp.float32) + bmu_ref[...]
    lv = jnp.dot(h2, wlv_ref[...], preferred_element_type=jnp.float32) + blv_ref[...]
    z = mu + eps_ref[...] * jnp.exp(0.5 * lv)
    z_ref[...] = z
    mu_ref[...] = mu
    lv_ref[...] = lv

    row_ids = pid * BN + lax.broadcasted_iota(jnp.int32, (BN, 1), 0)
    valid = row_ids < N
    giota = lax.broadcasted_iota(jnp.int32, (BN, G), 1)
    onehot = jnp.where(jnp.logical_and(valid, batch_ref[...] == giota), 1.0, 0.0)
    z_safe = jnp.where(valid, z, 0.0)
    psum[...] += lax.dot_general(onehot, z_safe, (((0,), (0,)), ((), ())),
                                 preferred_element_type=jnp.float32)
    pcnt[...] += lax.dot_general(onehot, jnp.ones((BN, 1), jnp.float32),
                                 (((0,), (0,)), ((), ())),
                                 preferred_element_type=jnp.float32)

    @pl.when(pid == NBLK - 1)
    def _():
        emb = psum[...] / jnp.clip(pcnt[...], 1.0, None)
        logits_ref[...] = jnp.dot(emb, wc_ref[...],
                                  preferred_element_type=jnp.float32) + bc_ref[...]


def _mlp2(h1q, agg2q, batch2, eps_noise,
          W2a, b2a, W2b, b2b, Wmu, bmu, Wlv, blv, Wc, bc):
    full = lambda shape: pl.BlockSpec(shape, lambda i: tuple(0 for _ in shape))
    return pl.pallas_call(
        _mlp2_body,
        grid=(NBLK,),
        in_specs=[pl.BlockSpec((BN, QQ), lambda i: (i, 0)) for _ in range(8)]
        + [
            pl.BlockSpec((BN, 1), lambda i: (i, 0)),
            pl.BlockSpec((BN, L), lambda i: (i, 0)),
            full((H, H)), full((1, H)), full((H, H)), full((1, H)),
            full((H, L)), full((1, L)), full((H, L)), full((1, L)),
            full((L, C)), full((1, C)),
        ],
        out_specs=[
            pl.BlockSpec((BN, L), lambda i: (i, 0)),
            pl.BlockSpec((BN, L), lambda i: (i, 0)),
            pl.BlockSpec((BN, L), lambda i: (i, 0)),
            pl.BlockSpec((G, C), lambda i: (0, 0)),
        ],
        out_shape=[
            jax.ShapeDtypeStruct((N, L), jnp.float32),
            jax.ShapeDtypeStruct((N, L), jnp.float32),
            jax.ShapeDtypeStruct((N, L), jnp.float32),
            jax.ShapeDtypeStruct((G, C), jnp.float32),
        ],
        scratch_shapes=[
            pltpu.VMEM((G, L), jnp.float32),
            pltpu.VMEM((G, 1), jnp.float32),
        ],
    )(*h1q, *agg2q, batch2, eps_noise,
      W2a, b2a.reshape(1, H), W2b, b2b.reshape(1, H),
      Wmu, bmu.reshape(1, L), Wlv, blv.reshape(1, L),
      Wc, bc.reshape(1, C))


def kernel(x, edge_index, edge_attr, batch, We1, be1, W1a, b1a, W1b, b1b,
           We2, be2, W2a, b2a, W2b, b2b, Wmu, bmu, Wlv, blv, Wc, bc, eps_noise):
    src = edge_index[0]
    dst = edge_index[1]

    e1_2d, *e2q = _edge_proj(edge_attr.T, We1, be1, We2, be2)
    e1flat = e1_2d.reshape(E)

    agg1p = _sc_layer1(x, src, dst, e1flat)

    h1q = _mlp1(x.reshape(N, 1), agg1p.reshape(2, NPAD1, 1),
                W1a, b1a, W1b, b1b)

    agg2q = _sc_layer2(h1q, src, dst, e2q)

    z, mu, lv, logits = _mlp2(h1q, agg2q, batch.reshape(N, 1),
                              eps_noise, W2a, b2a, W2b, b2b,
                              Wmu, bmu, Wlv, blv, Wc, bc)
    return (z, mu, lv, logits)


# speedup vs baseline: 1.1631x; 1.1631x over previous
import functools

import jax
import jax.numpy as jnp
from jax import lax
from jax.experimental import pallas as pl
from jax.experimental.pallas import tpu as pltpu
from jax.experimental.pallas import tpu_sc as plsc

N = 50000
E = 1600000
EA = 16
H = 64
HH = 32
L = 32
C = 6
G = 128

BE = 2560
NBLKA = E // BE


QQ = 16


def _eproj_body(eat_ref, w1t_ref, b1_ref, w2_ref, b2_ref, e1_ref, *e2q_refs):
    eat = eat_ref[...]
    e2 = lax.dot_general(eat, w2_ref[...], (((0,), (0,)), ((), ())),
                         preferred_element_type=jnp.float32) + b2_ref[...]
    for q in range(4):
        e2q_refs[q][...] = e2[:, q * QQ:(q + 1) * QQ]
    e1 = lax.dot_general(w1t_ref[...], eat, (((1,), (0,)), ((), ())),
                         preferred_element_type=jnp.float32) + b1_ref[...]
    e1_ref[...] = e1.reshape(1, 1, BE)


def _edge_proj(eat, We1, be1, We2, be2):
    return pl.pallas_call(
        _eproj_body,
        grid=(NBLKA,),
        in_specs=[
            pl.BlockSpec((EA, BE), lambda i: (0, i)),
            pl.BlockSpec((1, EA), lambda i: (0, 0)),
            pl.BlockSpec((1, 1), lambda i: (0, 0)),
            pl.BlockSpec((EA, H), lambda i: (0, 0)),
            pl.BlockSpec((1, H), lambda i: (0, 0)),
        ],
        out_specs=[pl.BlockSpec((1, 1, BE), lambda i: (i, 0, 0))]
        + [pl.BlockSpec((BE, QQ), lambda i: (i, 0)) for _ in range(4)],
        out_shape=[jax.ShapeDtypeStruct((NBLKA, 1, BE), jnp.float32)]
        + [jax.ShapeDtypeStruct((E, QQ), jnp.float32) for _ in range(4)],
    )(eat, We1.reshape(1, EA), be1.reshape(1, 1), We2, be2.reshape(1, H))


CH1 = 2560
NCH1 = E // CH1
ZPT1 = 3200
NPAD1 = 16 * ZPT1


def _sc_layer1(x, src, dst, e1flat):
    mesh = plsc.VectorSubcoreMesh(core_axis_name="c", subcore_axis_name="s")

    @functools.partial(
        pl.kernel,
        out_type=jax.ShapeDtypeStruct((2, NPAD1), jnp.float32),
        mesh=mesh,
        scratch_types=[
            pltpu.VMEM((N,), jnp.float32),
            pltpu.VMEM((CH1,), jnp.int32),
            pltpu.VMEM((CH1,), jnp.int32),
            pltpu.VMEM((CH1,), jnp.float32),
            pltpu.VMEM((CH1,), jnp.float32),
            pltpu.VMEM((ZPT1,), jnp.float32),
            pltpu.VMEM_SHARED((NPAD1,), jnp.float32),
            pltpu.SemaphoreType.DMA,
        ],
        compiler_params=pltpu.CompilerParams(needs_layout_passes=False, use_tc_tiling_on_sc=False),
    )
    def bkern(x_hbm, src_hbm, dst_hbm, e1_hbm, out_hbm,
              x_v, src_v, dst_v, e1_v, msg_v, zb_v, acc_s, sem):
        c = lax.axis_index("c")
        s = lax.axis_index("s")
        w = s * 2 + c
        pltpu.sync_copy(x_hbm, x_v)

        def zrow(r, _):
            zb_v[pl.ds(r * 16, 16)] = jnp.zeros((16,), jnp.float32)
            return 0

        lax.fori_loop(0, ZPT1 // 16, zrow, 0, unroll=4)
        pltpu.sync_copy(zb_v, acc_s.at[pl.ds(s * ZPT1, ZPT1)])
        plsc.subcore_barrier()

        def chunk(j, _):
            t = w + j * 32

            @pl.when(t < NCH1)
            def _():
                base = t * CH1
                pltpu.sync_copy(src_hbm.at[pl.ds(base, CH1)], src_v)
                pltpu.sync_copy(dst_hbm.at[pl.ds(base, CH1)], dst_v)
                pltpu.sync_copy(e1_hbm.at[pl.ds(base, CH1)], e1_v)

                def row(r, _):
                    for k in range(5):
                        sl = pl.ds((r * 5 + k) * 16, 16)
                        g = plsc.load_gather(x_v, [src_v[sl]])
                        msg_v[sl] = jnp.maximum(g + e1_v[sl], 0.0)
                    return 0

                lax.fori_loop(0, CH1 // 80, row, 0, unroll=4)
                pltpu.sync_copy(msg_v, acc_s.at[dst_v], add=True)

            return 0

        lax.fori_loop(0, (NCH1 + 31) // 32, chunk, 0, unroll=1)
        plsc.subcore_barrier()

        lo = s * ZPT1
        pltpu.sync_copy(acc_s.at[pl.ds(lo, ZPT1)], x_v.at[pl.ds(0, ZPT1)])

        @pl.when(c == 0)
        def _():
            pltpu.sync_copy(x_v.at[pl.ds(0, ZPT1)], out_hbm.at[0].at[pl.ds(lo, ZPT1)])

        @pl.when(c == 1)
        def _():
            pltpu.sync_copy(x_v.at[pl.ds(0, ZPT1)], out_hbm.at[1].at[pl.ds(lo, ZPT1)])

    return bkern(x, src, dst, e1flat)


BN = 2048
NBLK = (N + BN - 1) // BN


def _mlp1_body(x_ref, a_ref, w1a_ref, b1a_ref, w1b_ref, b1b_ref, *h1q_refs):
    h = x_ref[...] + a_ref[0] + a_ref[1]
    t = jnp.maximum(h * w1a_ref[...] + b1a_ref[...], 0.0)
    h1 = jnp.dot(t, w1b_ref[...], preferred_element_type=jnp.float32) + b1b_ref[...]
    h1 = jnp.maximum(h1, 0.0)
    for q in range(4):
        h1q_refs[q][...] = h1[:, q * QQ:(q + 1) * QQ]


def _mlp1(x2, agg1p3, W1a, b1a, W1b, b1b):
    return pl.pallas_call(
        _mlp1_body,
        grid=(NBLK,),
        in_specs=[
            pl.BlockSpec((BN, 1), lambda i: (i, 0)),
            pl.BlockSpec((2, BN, 1), lambda i: (0, i, 0)),
            pl.BlockSpec((1, H), lambda i: (0, 0)),
            pl.BlockSpec((1, H), lambda i: (0, 0)),
            pl.BlockSpec((H, H), lambda i: (0, 0)),
            pl.BlockSpec((1, H), lambda i: (0, 0)),
        ],
        out_specs=[pl.BlockSpec((BN, QQ), lambda i: (i, 0)) for _ in range(4)],
        out_shape=[jax.ShapeDtypeStruct((N, QQ), jnp.float32) for _ in range(4)],
    )(x2, agg1p3, W1a, b1a.reshape(1, H), W1b, b1b.reshape(1, H))


CH2 = 1280
NCH2 = E // CH2
RPT = 3200
NPAD2 = 16 * RPT


def _sc_layer2(h1q, src, dst, e2q):
    mesh = plsc.VectorSubcoreMesh(core_axis_name="c", subcore_axis_name="s")

    @functools.partial(
        pl.kernel,
        out_type=[jax.ShapeDtypeStruct((NPAD2, QQ), jnp.float32)
                  for _ in range(4)],
        mesh=mesh,
        scratch_types=[
            pltpu.VMEM((CH2,), jnp.int32),
            pltpu.VMEM((CH2,), jnp.int32),
            pltpu.VMEM((CH2, QQ), jnp.float32),
            pltpu.VMEM((CH2, QQ), jnp.float32),
            pltpu.VMEM_SHARED((NPAD2, QQ), jnp.float32),
            pltpu.SemaphoreType.DMA,
        ],
        compiler_params=pltpu.CompilerParams(needs_layout_passes=False,
                                             use_tc_tiling_on_sc=False),
    )
    def dkern(h0_hbm, h1_hbm, h2_hbm, h3_hbm, src_hbm, dst_hbm,
              e0_hbm, e1_hbm, e2_hbm, e3_hbm, o0_hbm, o1_hbm, o2_hbm, o3_hbm,
              src_v, dst_v, e2_v, gath_v, acc_s, sem):
        c = lax.axis_index("c")
        s = lax.axis_index("s")
        hq = (h0_hbm, h1_hbm, h2_hbm, h3_hbm)
        eq = (e0_hbm, e1_hbm, e2_hbm, e3_hbm)
        oq = (o0_hbm, o1_hbm, o2_hbm, o3_hbm)

        def zero_acc():
            def zrow(r, _):
                e2_v[r, pl.ds(0, 16)] = jnp.zeros((16,), jnp.float32)
                return 0

            lax.fori_loop(0, CH2, zrow, 0, unroll=4)
            for k in range(2):
                pltpu.sync_copy(e2_v, acc_s.at[pl.ds(s * RPT + k * CH2, CH2)])
            pltpu.sync_copy(e2_v.at[pl.ds(0, RPT - 2 * CH2)],
                            acc_s.at[pl.ds(s * RPT + 2 * CH2, RPT - 2 * CH2)])

        def qpass(h1_hbm, e2_hbm):
            def chunk(j, _):
                t = s + j * 16

                @pl.when(t < NCH2)
                def _():
                    base = t * CH2
                    pltpu.sync_copy(src_hbm.at[pl.ds(base, CH2)], src_v)
                    pltpu.sync_copy(dst_hbm.at[pl.ds(base, CH2)], dst_v)
                    pltpu.sync_copy(e2_hbm.at[pl.ds(base, CH2)], e2_v)
                    pltpu.async_copy(h1_hbm.at[src_v], gath_v, sem).wait()

                    def row(r, _):
                        for k in range(4):
                            rr = r * 4 + k
                            a0 = gath_v[rr, pl.ds(0, 16)] + e2_v[rr, pl.ds(0, 16)]
                            gath_v[rr, pl.ds(0, 16)] = jnp.maximum(a0, 0.0)
                        return 0

                    lax.fori_loop(0, CH2 // 4, row, 0, unroll=1)
                    pltpu.sync_copy(gath_v, acc_s.at[dst_v], add=True)

                return 0

            lax.fori_loop(0, (NCH2 + 15) // 16, chunk, 0, unroll=1)

        def copyout(out_hbm):
            for k in range(3):
                ln = CH2 if k < 2 else RPT - 2 * CH2
                off = s * RPT + k * CH2
                pltpu.sync_copy(acc_s.at[pl.ds(off, ln)], gath_v.at[pl.ds(0, ln)])
                pltpu.sync_copy(gath_v.at[pl.ds(0, ln)], out_hbm.at[pl.ds(off, ln)])

        for p in range(2):
            zero_acc()
            plsc.subcore_barrier()

            @pl.when(c == 0)
            def _():
                qpass(hq[p], eq[p])

            @pl.when(c == 1)
            def _():
                qpass(hq[2 + p], eq[2 + p])

            plsc.subcore_barrier()

            @pl.when(c == 0)
            def _():
                copyout(oq[p])

            @pl.when(c == 1)
            def _():
                copyout(oq[2 + p])

            if p == 0:
                plsc.subcore_barrier()

    return dkern(*h1q, src, dst, *e2q)


def _mlp2_body(h0_ref, h1_ref, h2q_ref, h3_ref, a0_ref, a1_ref, a2_ref, a3_ref,
               batch_ref, eps_ref,
               w2a_ref, b2a_ref, w2b_ref, b2b_ref, wmu_ref, bmu_ref,
               wlv_ref, blv_ref, wc_ref, bc_ref,
               z_ref, mu_ref, lv_ref, logits_ref, psum, pcnt):
    pid = pl.program_id(0)

    @pl.when(pid == 0)
    def _():
        psum[...] = jnp.zeros_like(psum)
        pcnt[...] = jnp.zeros_like(pcnt)

    h1 = jnp.concatenate([h0_ref[...], h1_ref[...], h2q_ref[...], h3_ref[...]],
                         axis=1)
    agg = jnp.concatenate([a0_ref[...], a1_ref[...], a2_ref[...], a3_ref[...]],
                          axis=1)
    h = h1 + agg
    t = jnp.maximum(jnp.dot(h, w2a_ref[...], preferred_element_type=jnp.float32)
                    + b2a_ref[...], 0.0)
    h2 = jnp.dot(t, w2b_ref[...], preferred_element_type=jnp.float32) + b2b_ref[...]
    h2 = jnp.maximum(h2, 0.0)
    mu = jnp.dot(h2, wmu_ref[...], preferred_element_type=jnp.float32) + bmu_ref[...]
    lv = jnp.dot(h2, wlv_ref[...], preferred_element_type=jnp.float32) + blv_ref[...]
    z = mu + eps_ref[...] * jnp.exp(0.5 * lv)
    z_ref[...] = z
    mu_ref[...] = mu
    lv_ref[...] = lv

    row_ids = pid * BN + lax.broadcasted_iota(jnp.int32, (BN, 1), 0)
    valid = row_ids < N
    giota = lax.broadcasted_iota(jnp.int32, (BN, G), 1)
    onehot = jnp.where(jnp.logical_and(valid, batch_ref[...] == giota), 1.0, 0.0)
    z_safe = jnp.where(valid, z, 0.0)
    psum[...] += lax.dot_general(onehot, z_safe, (((0,), (0,)), ((), ())),
                                 preferred_element_type=jnp.float32)
    pcnt[...] += lax.dot_general(onehot, jnp.ones((BN, 1), jnp.float32),
                                 (((0,), (0,)), ((), ())),
                                 preferred_element_type=jnp.float32)

    @pl.when(pid == NBLK - 1)
    def _():
        emb = psum[...] / jnp.clip(pcnt[...], 1.0, None)
        logits_ref[...] = jnp.dot(emb, wc_ref[...],
                                  preferred_element_type=jnp.float32) + bc_ref[...]


def _mlp2(h1q, agg2q, batch2, eps_noise,
          W2a, b2a, W2b, b2b, Wmu, bmu, Wlv, blv, Wc, bc):
    full = lambda shape: pl.BlockSpec(shape, lambda i: tuple(0 for _ in shape))
    return pl.pallas_call(
        _mlp2_body,
        grid=(NBLK,),
        in_specs=[pl.BlockSpec((BN, QQ), lambda i: (i, 0)) for _ in range(8)]
        + [
            pl.BlockSpec((BN, 1), lambda i: (i, 0)),
            pl.BlockSpec((BN, L), lambda i: (i, 0)),
            full((H, H)), full((1, H)), full((H, H)), full((1, H)),
            full((H, L)), full((1, L)), full((H, L)), full((1, L)),
            full((L, C)), full((1, C)),
        ],
        out_specs=[
            pl.BlockSpec((BN, L), lambda i: (i, 0)),
            pl.BlockSpec((BN, L), lambda i: (i, 0)),
            pl.BlockSpec((BN, L), lambda i: (i, 0)),
            pl.BlockSpec((G, C), lambda i: (0, 0)),
        ],
        out_shape=[
            jax.ShapeDtypeStruct((N, L), jnp.float32),
            jax.ShapeDtypeStruct((N, L), jnp.float32),
            jax.ShapeDtypeStruct((N, L), jnp.float32),
            jax.ShapeDtypeStruct((G, C), jnp.float32),
        ],
        scratch_shapes=[
            pltpu.VMEM((G, L), jnp.float32),
            pltpu.VMEM((G, 1), jnp.float32),
        ],
    )(*h1q, *agg2q, batch2, eps_noise,
      W2a, b2a.reshape(1, H), W2b, b2b.reshape(1, H),
      Wmu, bmu.reshape(1, L), Wlv, blv.reshape(1, L),
      Wc, bc.reshape(1, C))


def kernel(x, edge_index, edge_attr, batch, We1, be1, W1a, b1a, W1b, b1b,
           We2, be2, W2a, b2a, W2b, b2b, Wmu, bmu, Wlv, blv, Wc, bc, eps_noise):
    src = edge_index[0]
    dst = edge_index[1]

    e1_2d, *e2q = _edge_proj(edge_attr.T, We1, be1, We2, be2)
    e1flat = e1_2d.reshape(E)

    agg1p = _sc_layer1(x, src, dst, e1flat)

    h1q = _mlp1(x.reshape(N, 1), agg1p.reshape(2, NPAD1, 1),
                W1a, b1a, W1b, b1b)

    agg2q = _sc_layer2(h1q, src, dst, e2q)

    z, mu, lv, logits = _mlp2(h1q, agg2q, batch.reshape(N, 1),
                              eps_noise, W2a, b2a, W2b, b2b,
                              Wmu, bmu, Wlv, blv, Wc, bc)
    return (z, mu, lv, logits)


# speedup vs baseline: 1.2658x; 1.0882x over previous
import functools

import jax
import jax.numpy as jnp
from jax import lax
from jax.experimental import pallas as pl
from jax.experimental.pallas import tpu as pltpu
from jax.experimental.pallas import tpu_sc as plsc

N = 50000
E = 1600000
EA = 16
H = 64
HH = 32
L = 32
C = 6
G = 128

BE = 2560
NBLKA = E // BE


QQ = 16


def _eproj_body(eat_ref, w1t_ref, b1_ref, w2_ref, b2_ref, e1_ref, *e2q_refs):
    eat = eat_ref[...]
    e2 = lax.dot_general(eat, w2_ref[...], (((0,), (0,)), ((), ())),
                         preferred_element_type=jnp.float32) + b2_ref[...]
    for q in range(4):
        e2q_refs[q][...] = e2[:, q * QQ:(q + 1) * QQ]
    e1 = lax.dot_general(w1t_ref[...], eat, (((1,), (0,)), ((), ())),
                         preferred_element_type=jnp.float32) + b1_ref[...]
    e1_ref[...] = e1.reshape(1, 1, BE)


def _edge_proj(eat, We1, be1, We2, be2):
    return pl.pallas_call(
        _eproj_body,
        grid=(NBLKA,),
        in_specs=[
            pl.BlockSpec((EA, BE), lambda i: (0, i)),
            pl.BlockSpec((1, EA), lambda i: (0, 0)),
            pl.BlockSpec((1, 1), lambda i: (0, 0)),
            pl.BlockSpec((EA, H), lambda i: (0, 0)),
            pl.BlockSpec((1, H), lambda i: (0, 0)),
        ],
        out_specs=[pl.BlockSpec((1, 1, BE), lambda i: (i, 0, 0))]
        + [pl.BlockSpec((BE, QQ), lambda i: (i, 0)) for _ in range(4)],
        out_shape=[jax.ShapeDtypeStruct((NBLKA, 1, BE), jnp.float32)]
        + [jax.ShapeDtypeStruct((E, QQ), jnp.float32) for _ in range(4)],
    )(eat, We1.reshape(1, EA), be1.reshape(1, 1), We2, be2.reshape(1, H))


CH1 = 2560
NCH1 = E // CH1
ZPT1 = 3200
NPAD1 = 16 * ZPT1


def _sc_layer1(x, src, dst, e1flat):
    mesh = plsc.VectorSubcoreMesh(core_axis_name="c", subcore_axis_name="s")

    @functools.partial(
        pl.kernel,
        out_type=jax.ShapeDtypeStruct((2, NPAD1), jnp.float32),
        mesh=mesh,
        scratch_types=[
            pltpu.VMEM((N,), jnp.float32),
            pltpu.VMEM((CH1,), jnp.int32),
            pltpu.VMEM((CH1,), jnp.int32),
            pltpu.VMEM((CH1,), jnp.float32),
            pltpu.VMEM((CH1,), jnp.float32),
            pltpu.VMEM((ZPT1,), jnp.float32),
            pltpu.VMEM_SHARED((NPAD1,), jnp.float32),
            pltpu.SemaphoreType.DMA,
        ],
        compiler_params=pltpu.CompilerParams(needs_layout_passes=False, use_tc_tiling_on_sc=False),
    )
    def bkern(x_hbm, src_hbm, dst_hbm, e1_hbm, out_hbm,
              x_v, src_v, dst_v, e1_v, msg_v, zb_v, acc_s, sem):
        c = lax.axis_index("c")
        s = lax.axis_index("s")
        w = s * 2 + c
        pltpu.sync_copy(x_hbm, x_v)

        def zrow(r, _):
            zb_v[pl.ds(r * 16, 16)] = jnp.zeros((16,), jnp.float32)
            return 0

        lax.fori_loop(0, ZPT1 // 16, zrow, 0, unroll=4)
        pltpu.sync_copy(zb_v, acc_s.at[pl.ds(s * ZPT1, ZPT1)])
        plsc.subcore_barrier()

        def chunk(j, _):
            t = w + j * 32

            @pl.when(t < NCH1)
            def _():
                base = t * CH1
                pltpu.sync_copy(src_hbm.at[pl.ds(base, CH1)], src_v)
                pltpu.sync_copy(dst_hbm.at[pl.ds(base, CH1)], dst_v)
                pltpu.sync_copy(e1_hbm.at[pl.ds(base, CH1)], e1_v)

                def row(r, _):
                    for k in range(5):
                        sl = pl.ds((r * 5 + k) * 16, 16)
                        g = plsc.load_gather(x_v, [src_v[sl]])
                        msg_v[sl] = jnp.maximum(g + e1_v[sl], 0.0)
                    return 0

                lax.fori_loop(0, CH1 // 80, row, 0, unroll=4)
                pltpu.sync_copy(msg_v, acc_s.at[dst_v], add=True)

            return 0

        lax.fori_loop(0, (NCH1 + 31) // 32, chunk, 0, unroll=1)
        plsc.subcore_barrier()

        lo = s * ZPT1
        pltpu.sync_copy(acc_s.at[pl.ds(lo, ZPT1)], x_v.at[pl.ds(0, ZPT1)])

        @pl.when(c == 0)
        def _():
            pltpu.sync_copy(x_v.at[pl.ds(0, ZPT1)], out_hbm.at[0].at[pl.ds(lo, ZPT1)])

        @pl.when(c == 1)
        def _():
            pltpu.sync_copy(x_v.at[pl.ds(0, ZPT1)], out_hbm.at[1].at[pl.ds(lo, ZPT1)])

    return bkern(x, src, dst, e1flat)


BN = 2048
NBLK = (N + BN - 1) // BN


def _mlp1_body(x_ref, a_ref, w1a_ref, b1a_ref, w1b_ref, b1b_ref, *h1q_refs):
    h = x_ref[...] + a_ref[0] + a_ref[1]
    t = jnp.maximum(h * w1a_ref[...] + b1a_ref[...], 0.0)
    h1 = jnp.dot(t, w1b_ref[...], preferred_element_type=jnp.float32) + b1b_ref[...]
    h1 = jnp.maximum(h1, 0.0)
    for q in range(4):
        h1q_refs[q][...] = h1[:, q * QQ:(q + 1) * QQ]


def _mlp1(x2, agg1p3, W1a, b1a, W1b, b1b):
    return pl.pallas_call(
        _mlp1_body,
        grid=(NBLK,),
        in_specs=[
            pl.BlockSpec((BN, 1), lambda i: (i, 0)),
            pl.BlockSpec((2, BN, 1), lambda i: (0, i, 0)),
            pl.BlockSpec((1, H), lambda i: (0, 0)),
            pl.BlockSpec((1, H), lambda i: (0, 0)),
            pl.BlockSpec((H, H), lambda i: (0, 0)),
            pl.BlockSpec((1, H), lambda i: (0, 0)),
        ],
        out_specs=[pl.BlockSpec((BN, QQ), lambda i: (i, 0)) for _ in range(4)],
        out_shape=[jax.ShapeDtypeStruct((N, QQ), jnp.float32) for _ in range(4)],
    )(x2, agg1p3, W1a, b1a.reshape(1, H), W1b, b1b.reshape(1, H))


CH2 = 640
NCH2 = E // CH2
NJ = NCH2 // 16
RPT = 3200
NPAD2 = 16 * RPT


def _sc_layer2(h1q, src, dst, e2q):
    mesh = plsc.VectorSubcoreMesh(core_axis_name="c", subcore_axis_name="s")

    @functools.partial(
        pl.kernel,
        out_type=[jax.ShapeDtypeStruct((NPAD2, QQ), jnp.float32)
                  for _ in range(4)],
        mesh=mesh,
        scratch_types=[
            pltpu.VMEM((CH2,), jnp.int32),
            pltpu.VMEM((CH2,), jnp.int32),
            pltpu.VMEM((CH2,), jnp.int32),
            pltpu.VMEM((CH2,), jnp.int32),
            pltpu.VMEM((CH2, QQ), jnp.float32),
            pltpu.VMEM((CH2, QQ), jnp.float32),
            pltpu.VMEM((CH2, QQ), jnp.float32),
            pltpu.VMEM((CH2, QQ), jnp.float32),
            pltpu.VMEM_SHARED((NPAD2, QQ), jnp.float32),
            pltpu.SemaphoreType.DMA,
            pltpu.SemaphoreType.DMA,
            pltpu.SemaphoreType.DMA,
            pltpu.SemaphoreType.DMA,
        ],
        compiler_params=pltpu.CompilerParams(needs_layout_passes=False,
                                             use_tc_tiling_on_sc=False),
    )
    def dkern(h0_hbm, h1_hbm, h2_hbm, h3_hbm, src_hbm, dst_hbm,
              e0_hbm, e1_hbm, e2_hbm, e3_hbm, o0_hbm, o1_hbm, o2_hbm, o3_hbm,
              src_v0, src_v1, dst_v0, dst_v1, e2_v0, e2_v1, gath_v0, gath_v1,
              acc_s, semL0, semL1, semG0, semG1):
        c = lax.axis_index("c")
        s = lax.axis_index("s")
        hq = (h0_hbm, h1_hbm, h2_hbm, h3_hbm)
        eq = (e0_hbm, e1_hbm, e2_hbm, e3_hbm)
        oq = (o0_hbm, o1_hbm, o2_hbm, o3_hbm)
        src_v = (src_v0, src_v1)
        dst_v = (dst_v0, dst_v1)
        e2_v = (e2_v0, e2_v1)
        gath_v = (gath_v0, gath_v1)
        semL = (semL0, semL1)
        semG = (semG0, semG1)

        def zero_acc():
            def zrow(r, _):
                e2_v0[r, pl.ds(0, 16)] = jnp.zeros((16,), jnp.float32)
                return 0

            lax.fori_loop(0, CH2, zrow, 0, unroll=4)
            for k in range(RPT // CH2):
                pltpu.sync_copy(e2_v0, acc_s.at[pl.ds(s * RPT + k * CH2, CH2)])

        def qpass(h1_hbm, e2_hbm):
            def loads(b, t):
                base = t * CH2
                pltpu.async_copy(src_hbm.at[pl.ds(base, CH2)], src_v[b], semL[b])
                pltpu.async_copy(dst_hbm.at[pl.ds(base, CH2)], dst_v[b], semL[b])
                pltpu.async_copy(e2_hbm.at[pl.ds(base, CH2)], e2_v[b], semL[b])

            def wait_loads(b):
                pltpu.make_async_copy(src_hbm.at[pl.ds(0, CH2)], src_v[b], semL[b]).wait()
                pltpu.make_async_copy(dst_hbm.at[pl.ds(0, CH2)], dst_v[b], semL[b]).wait()
                pltpu.make_async_copy(e2_hbm.at[pl.ds(0, CH2)], e2_v[b], semL[b]).wait()

            def do_gather(b):
                pltpu.async_copy(h1_hbm.at[src_v[b]], gath_v[b], semG[b])

            def wait_gather(b):
                pltpu.make_async_copy(h1_hbm.at[pl.ds(0, CH2)], gath_v[b], semG[b]).wait()

            def compute(b):
                def row(r, _):
                    for k in range(4):
                        rr = r * 4 + k
                        a0 = gath_v[b][rr, pl.ds(0, 16)] + e2_v[b][rr, pl.ds(0, 16)]
                        gath_v[b][rr, pl.ds(0, 16)] = jnp.maximum(a0, 0.0)
                    return 0

                lax.fori_loop(0, CH2 // 4, row, 0, unroll=1)

            def scatter(b):
                pltpu.sync_copy(gath_v[b], acc_s.at[dst_v[b]], add=True)

            loads(0, s)
            wait_loads(0)
            do_gather(0)
            loads(1, s + 16)

            def superstep(u, _):
                for b in range(2):
                    j = u * 2 + b
                    wait_loads(1 - b)
                    wait_gather(b)
                    do_gather(1 - b)
                    compute(b)
                    scatter(b)
                    loads(b, s + (j + 2) * 16)
                return 0

            lax.fori_loop(0, (NJ - 2) // 2, superstep, 0, unroll=1)
            wait_loads(1)
            wait_gather(0)
            do_gather(1)
            compute(0)
            scatter(0)
            wait_gather(1)
            compute(1)
            scatter(1)

            @pl.when(s < NCH2 - 16 * NJ)
            def _():
                t = 16 * NJ + s
                loads(0, t)
                wait_loads(0)
                do_gather(0)
                wait_gather(0)
                compute(0)
                scatter(0)

        def copyout(out_hbm):
            for k in range(RPT // CH2):
                off = s * RPT + k * CH2
                pltpu.sync_copy(acc_s.at[pl.ds(off, CH2)], gath_v0)
                pltpu.sync_copy(gath_v0, out_hbm.at[pl.ds(off, CH2)])

        for p in range(2):
            zero_acc()
            plsc.subcore_barrier()

            @pl.when(c == 0)
            def _():
                qpass(hq[p], eq[p])

            @pl.when(c == 1)
            def _():
                qpass(hq[2 + p], eq[2 + p])

            plsc.subcore_barrier()

            @pl.when(c == 0)
            def _():
                copyout(oq[p])

            @pl.when(c == 1)
            def _():
                copyout(oq[2 + p])

            if p == 0:
                plsc.subcore_barrier()

    return dkern(*h1q, src, dst, *e2q)


def _mlp2_body(h0_ref, h1_ref, h2q_ref, h3_ref, a0_ref, a1_ref, a2_ref, a3_ref,
               batch_ref, eps_ref,
               w2a_ref, b2a_ref, w2b_ref, b2b_ref, wmu_ref, bmu_ref,
               wlv_ref, blv_ref, wc_ref, bc_ref,
               z_ref, mu_ref, lv_ref, logits_ref, psum, pcnt):
    pid = pl.program_id(0)

    @pl.when(pid == 0)
    def _():
        psum[...] = jnp.zeros_like(psum)
        pcnt[...] = jnp.zeros_like(pcnt)

    h1 = jnp.concatenate([h0_ref[...], h1_ref[...], h2q_ref[...], h3_ref[...]],
                         axis=1)
    agg = jnp.concatenate([a0_ref[...], a1_ref[...], a2_ref[...], a3_ref[...]],
                          axis=1)
    h = h1 + agg
    t = jnp.maximum(jnp.dot(h, w2a_ref[...], preferred_element_type=jnp.float32)
                    + b2a_ref[...], 0.0)
    h2 = jnp.dot(t, w2b_ref[...], preferred_element_type=jnp.float32) + b2b_ref[...]
    h2 = jnp.maximum(h2, 0.0)
    mu = jnp.dot(h2, wmu_ref[...], preferred_element_type=jnp.float32) + bmu_ref[...]
    lv = jnp.dot(h2, wlv_ref[...], preferred_element_type=jnp.float32) + blv_ref[...]
    z = mu + eps_ref[...] * jnp.exp(0.5 * lv)
    z_ref[...] = z
    mu_ref[...] = mu
    lv_ref[...] = lv

    row_ids = pid * BN + lax.broadcasted_iota(jnp.int32, (BN, 1), 0)
    valid = row_ids < N
    giota = lax.broadcasted_iota(jnp.int32, (BN, G), 1)
    onehot = jnp.where(jnp.logical_and(valid, batch_ref[...] == giota), 1.0, 0.0)
    z_safe = jnp.where(valid, z, 0.0)
    psum[...] += lax.dot_general(onehot, z_safe, (((0,), (0,)), ((), ())),
                                 preferred_element_type=jnp.float32)
    pcnt[...] += lax.dot_general(onehot, jnp.ones((BN, 1), jnp.float32),
                                 (((0,), (0,)), ((), ())),
                                 preferred_element_type=jnp.float32)

    @pl.when(pid == NBLK - 1)
    def _():
        emb = psum[...] / jnp.clip(pcnt[...], 1.0, None)
        logits_ref[...] = jnp.dot(emb, wc_ref[...],
                                  preferred_element_type=jnp.float32) + bc_ref[...]


def _mlp2(h1q, agg2q, batch2, eps_noise,
          W2a, b2a, W2b, b2b, Wmu, bmu, Wlv, blv, Wc, bc):
    full = lambda shape: pl.BlockSpec(shape, lambda i: tuple(0 for _ in shape))
    return pl.pallas_call(
        _mlp2_body,
        grid=(NBLK,),
        in_specs=[pl.BlockSpec((BN, QQ), lambda i: (i, 0)) for _ in range(8)]
        + [
            pl.BlockSpec((BN, 1), lambda i: (i, 0)),
            pl.BlockSpec((BN, L), lambda i: (i, 0)),
            full((H, H)), full((1, H)), full((H, H)), full((1, H)),
            full((H, L)), full((1, L)), full((H, L)), full((1, L)),
            full((L, C)), full((1, C)),
        ],
        out_specs=[
            pl.BlockSpec((BN, L), lambda i: (i, 0)),
            pl.BlockSpec((BN, L), lambda i: (i, 0)),
            pl.BlockSpec((BN, L), lambda i: (i, 0)),
            pl.BlockSpec((G, C), lambda i: (0, 0)),
        ],
        out_shape=[
            jax.ShapeDtypeStruct((N, L), jnp.float32),
            jax.ShapeDtypeStruct((N, L), jnp.float32),
            jax.ShapeDtypeStruct((N, L), jnp.float32),
            jax.ShapeDtypeStruct((G, C), jnp.float32),
        ],
        scratch_shapes=[
            pltpu.VMEM((G, L), jnp.float32),
            pltpu.VMEM((G, 1), jnp.float32),
        ],
    )(*h1q, *agg2q, batch2, eps_noise,
      W2a, b2a.reshape(1, H), W2b, b2b.reshape(1, H),
      Wmu, bmu.reshape(1, L), Wlv, blv.reshape(1, L),
      Wc, bc.reshape(1, C))


def kernel(x, edge_index, edge_attr, batch, We1, be1, W1a, b1a, W1b, b1b,
           We2, be2, W2a, b2a, W2b, b2b, Wmu, bmu, Wlv, blv, Wc, bc, eps_noise):
    src = edge_index[0]
    dst = edge_index[1]

    e1_2d, *e2q = _edge_proj(edge_attr.T, We1, be1, We2, be2)
    e1flat = e1_2d.reshape(E)

    agg1p = _sc_layer1(x, src, dst, e1flat)

    h1q = _mlp1(x.reshape(N, 1), agg1p.reshape(2, NPAD1, 1),
                W1a, b1a, W1b, b1b)

    agg2q = _sc_layer2(h1q, src, dst, e2q)

    z, mu, lv, logits = _mlp2(h1q, agg2q, batch.reshape(N, 1),
                              eps_noise, W2a, b2a, W2b, b2b,
                              Wmu, bmu, Wlv, blv, Wc, bc)
    return (z, mu, lv, logits)
